# Initial kernel scaffold; baseline (speedup 1.0000x reference)
import jax
import jax.numpy as jnp
from jax.experimental import pallas as pl


def kernel(x, edge_index, W1, a1_src, a1_dst, b1, W2, a2_src, a2_dst, b2):
    raise NotImplementedError("write your pallas kernel here")


# speedup vs baseline: 15.1306x; 15.1306x over previous
import functools

import jax
import jax.numpy as jnp
import numpy as np
from jax import lax
from jax.experimental import pallas as pl
from jax.experimental.pallas import tpu as pltpu
from jax.experimental.pallas import tpu_sc as plsc

N = 10000
NPAD = 10240
E = 320000
FIN = 128
HID = 128
C = 8
L = 16
CE = 4000
NSLICE = E // 16
NEG_SLOPE = 0.2

_SC_PARAMS = pltpu.CompilerParams(needs_layout_passes=False)


def _mesh():
    return plsc.VectorSubcoreMesh(core_axis_name="c", subcore_axis_name="s")


def _i32(v):
    return lax.convert_element_type(v, jnp.int32)


def _fori(lo, hi, body):
    lax.fori_loop(jnp.int32(lo), jnp.int32(hi), body, jnp.int32(0))


_Z = np.int32(0)


def _lrelu(z):
    return jnp.where(z > 0, z, NEG_SLOPE * z)


def _dense1_body(x_ref, w_ref, avs_ref, avd_ref, ht_ref, as_ref, ad_ref):
    ht = lax.dot_general(w_ref[...], x_ref[...], (((0,), (1,)), ((), ())),
                         preferred_element_type=jnp.float32)
    ht_ref[...] = ht
    as_ref[...] = lax.dot_general(avs_ref[...], ht, (((0,), (0,)), ((), ())),
                                  preferred_element_type=jnp.float32)
    ad_ref[...] = lax.dot_general(avd_ref[...], ht, (((0,), (0,)), ((), ())),
                                  preferred_element_type=jnp.float32)


def _dense1(x_pad, W1, a1_src, a1_dst):
    BN = 512
    return pl.pallas_call(
        _dense1_body,
        grid=(NPAD // BN,),
        in_specs=[
            pl.BlockSpec((BN, FIN), lambda i: (i, _Z)),
            pl.BlockSpec((FIN, HID), lambda i: (_Z, _Z)),
            pl.BlockSpec((HID, 1), lambda i: (_Z, _Z)),
            pl.BlockSpec((HID, 1), lambda i: (_Z, _Z)),
        ],
        out_specs=[
            pl.BlockSpec((HID, BN), lambda i: (_Z, i)),
            pl.BlockSpec((1, BN), lambda i: (_Z, i)),
            pl.BlockSpec((1, BN), lambda i: (_Z, i)),
        ],
        out_shape=[
            jax.ShapeDtypeStruct((HID, NPAD), jnp.float32),
            jax.ShapeDtypeStruct((1, NPAD), jnp.float32),
            jax.ShapeDtypeStruct((1, NPAD), jnp.float32),
        ],
    )(x_pad, W1, a1_src.reshape(HID, 1), a1_dst.reshape(HID, 1))


def _table_max(tab):
    def body(i, m):
        return jnp.maximum(m, tab[pl.ds(i * L, L)])
    mvec = lax.fori_loop(jnp.int32(0), jnp.int32(NPAD // L), body,
                         jnp.full((L,), -1e30, jnp.float32))
    return jnp.max(mvec)


def _zero_1d(ref):
    def body(i, _):
        ref[pl.ds(i * L, L)] = jnp.zeros((L,), jnp.float32)
        return jnp.int32(0)
    _fori(0, NPAD // L, body)


def _denominators(sid, src_hbm, dst_hbm, as_t, ad_t, den_t, sbuf, dbuf,
                  stage, acc640, sp_part, smax):
    _zero_1d(den_t)

    def chunk(ci, _):
        off = sid * NSLICE + ci * CE
        pltpu.sync_copy(src_hbm.at[pl.ds(off, CE)], sbuf)
        pltpu.sync_copy(dst_hbm.at[pl.ds(off, CE)], dbuf)

        def it(i, _):
            sv = sbuf[pl.ds(i * L, L)]
            dv = dbuf[pl.ds(i * L, L)]
            a_s = plsc.load_gather(as_t, [sv])
            a_d = plsc.load_gather(ad_t, [dv])
            e = _lrelu(a_s + a_d)
            m = _lrelu(smax + a_d)
            plsc.addupdate_scatter(den_t, [dv], jnp.exp(e - m))
            return jnp.int32(0)

        _fori(0, CE // L, it)
        return jnp.int32(0)

    _fori(0, NSLICE // CE, chunk)

    CHK = NPAD // 16
    base = sid * CHK

    def addrows(lo, hi, first):
        def comb(k, _):
            pltpu.sync_copy(sp_part.at[k, pl.ds(base, CHK)], stage)

            def addit(i, _):
                acc640[pl.ds(i * L, L)] = (acc640[pl.ds(i * L, L)]
                                           + stage[pl.ds(i * L, L)])
                return jnp.int32(0)

            _fori(0, CHK // L, addit)
            return jnp.int32(0)

        if first:
            pltpu.sync_copy(sp_part.at[_i32(0), pl.ds(base, CHK)], acc640)
            _fori(1, hi, comb)
        else:
            _fori(lo, hi, comb)

    @pl.when(sid < 8)
    def _():
        pltpu.sync_copy(den_t, sp_part.at[sid])

    plsc.subcore_barrier()
    addrows(0, 8, True)
    plsc.subcore_barrier()

    @pl.when(sid >= 8)
    def _():
        pltpu.sync_copy(den_t, sp_part.at[sid - 8])

    plsc.subcore_barrier()
    addrows(0, 8, False)
    plsc.subcore_barrier()
    pltpu.sync_copy(acc640, sp_part.at[_i32(0), pl.ds(base, CHK)])
    plsc.subcore_barrier()
    pltpu.sync_copy(sp_part.at[_i32(0)], den_t)


def _edge_alpha(as_t, ad_t, den_t, sv, dv, smax):
    a_s = plsc.load_gather(as_t, [sv])
    a_d = plsc.load_gather(ad_t, [dv])
    den = plsc.load_gather(den_t, [dv])
    e = _lrelu(a_s + a_d)
    m = _lrelu(smax + a_d)
    return jnp.exp(e - m) / (den + 1e-16)


def _edges1_body(src_hbm, dst_hbm, ht_hbm, as_hbm, ad_hbm, o1t_hbm,
                 as_t, ad_t, den_t, tbl, acc, sbuf, dbuf, stage, acc640,
                 sp_part):
    cid = _i32(lax.axis_index("c"))
    sid = _i32(lax.axis_index("s"))
    tid = cid * 16 + sid

    pltpu.sync_copy(as_hbm, as_t)
    pltpu.sync_copy(ad_hbm, ad_t)
    smax = _table_max(as_t)

    _denominators(sid, src_hbm, dst_hbm, as_t, ad_t, den_t, sbuf, dbuf,
                  stage, acc640, sp_part, smax)

    pltpu.sync_copy(ht_hbm.at[pl.ds(tid * 4, 4)], tbl)

    jvs = [jnp.full((L,), j, jnp.int32) for j in range(4)]
    lanes = jnp.arange(L, dtype=jnp.int32)
    zvec = jnp.zeros((L,), jnp.float32)

    def zrow(i, _):
        cols = lanes + i * L
        for j in range(4):
            plsc.store_scatter(acc, [jvs[j], cols], zvec)
        return jnp.int32(0)

    _fori(0, NPAD // L, zrow)

    def chunk(ci, _):
        off = ci * CE
        pltpu.sync_copy(src_hbm.at[pl.ds(off, CE)], sbuf)
        pltpu.sync_copy(dst_hbm.at[pl.ds(off, CE)], dbuf)

        def it(i, _):
            sv = sbuf[pl.ds(i * L, L)]
            dv = dbuf[pl.ds(i * L, L)]
            alpha = _edge_alpha(as_t, ad_t, den_t, sv, dv, smax)
            for j in range(4):
                tv = plsc.load_gather(tbl, [jvs[j], sv])
                plsc.addupdate_scatter(acc, [jvs[j], dv], tv * alpha)
            return jnp.int32(0)

        _fori(0, CE // L, it)
        return jnp.int32(0)

    _fori(0, E // CE, chunk)
    pltpu.sync_copy(acc, o1t_hbm.at[pl.ds(tid * 4, 4)])


def _edges1(src, dst, ht, as_h, ad_h):
    return pl.kernel(
        _edges1_body,
        out_type=jax.ShapeDtypeStruct((HID, NPAD), jnp.float32),
        mesh=_mesh(),
        scratch_types=[
            pltpu.VMEM((NPAD,), jnp.float32),
            pltpu.VMEM((NPAD,), jnp.float32),
            pltpu.VMEM((NPAD,), jnp.float32),
            pltpu.VMEM((4, NPAD), jnp.float32),
            pltpu.VMEM((4, NPAD), jnp.float32),
            pltpu.VMEM((CE,), jnp.int32),
            pltpu.VMEM((CE,), jnp.int32),
            pltpu.VMEM((NPAD // 16,), jnp.float32),
            pltpu.VMEM((NPAD // 16,), jnp.float32),
            pltpu.VMEM_SHARED((8, NPAD), jnp.float32),
        ],
        compiler_params=_SC_PARAMS,
    )(src, dst, ht, as_h, ad_h)


def _dense2_body(o1t_ref, b1_ref, w2_ref, avs_ref, avd_ref,
                 h2t_ref, as2_ref, ad2_ref):
    h1 = jnp.maximum(o1t_ref[...] + b1_ref[...], 0.0)
    h2t = lax.dot_general(w2_ref[...], h1, (((0,), (0,)), ((), ())),
                          preferred_element_type=jnp.float32)
    h2t_ref[...] = h2t
    as2_ref[...] = lax.dot_general(avs_ref[...], h2t, (((0,), (0,)), ((), ())),
                                   preferred_element_type=jnp.float32)
    ad2_ref[...] = lax.dot_general(avd_ref[...], h2t, (((0,), (0,)), ((), ())),
                                   preferred_element_type=jnp.float32)


def _dense2(o1t, b1, W2, a2_src, a2_dst):
    BN = 512
    return pl.pallas_call(
        _dense2_body,
        grid=(NPAD // BN,),
        in_specs=[
            pl.BlockSpec((HID, BN), lambda i: (_Z, i)),
            pl.BlockSpec((HID, 1), lambda i: (_Z, _Z)),
            pl.BlockSpec((HID, C), lambda i: (_Z, _Z)),
            pl.BlockSpec((C, 1), lambda i: (_Z, _Z)),
            pl.BlockSpec((C, 1), lambda i: (_Z, _Z)),
        ],
        out_specs=[
            pl.BlockSpec((C, BN), lambda i: (_Z, i)),
            pl.BlockSpec((1, BN), lambda i: (_Z, i)),
            pl.BlockSpec((1, BN), lambda i: (_Z, i)),
        ],
        out_shape=[
            jax.ShapeDtypeStruct((C, NPAD), jnp.float32),
            jax.ShapeDtypeStruct((1, NPAD), jnp.float32),
            jax.ShapeDtypeStruct((1, NPAD), jnp.float32),
        ],
    )(o1t, b1.reshape(HID, 1), W2, a2_src.reshape(C, 1), a2_dst.reshape(C, 1))


EQ = E // 4


def _edges2_body(src_hbm, dst_hbm, h2t_hbm, as_hbm, ad_hbm, o2p_hbm,
                 as_t, ad_t, den_t, tbl, acc, sbuf, dbuf, stage, acc640,
                 sp_part):
    cid = _i32(lax.axis_index("c"))
    sid = _i32(lax.axis_index("s"))
    tid = cid * 16 + sid
    q = tid // C
    g = tid % C

    pltpu.sync_copy(as_hbm, as_t)
    pltpu.sync_copy(ad_hbm, ad_t)
    smax = _table_max(as_t)

    _denominators(sid, src_hbm, dst_hbm, as_t, ad_t, den_t, sbuf, dbuf,
                  stage, acc640, sp_part, smax)

    pltpu.sync_copy(h2t_hbm.at[g], tbl)
    _zero_1d(acc)

    def chunk(ci, _):
        off = q * EQ + ci * CE
        pltpu.sync_copy(src_hbm.at[pl.ds(off, CE)], sbuf)
        pltpu.sync_copy(dst_hbm.at[pl.ds(off, CE)], dbuf)

        def it(i, _):
            sv = sbuf[pl.ds(i * L, L)]
            dv = dbuf[pl.ds(i * L, L)]
            alpha = _edge_alpha(as_t, ad_t, den_t, sv, dv, smax)
            tv = plsc.load_gather(tbl, [sv])
            plsc.addupdate_scatter(acc, [dv], tv * alpha)
            return jnp.int32(0)

        _fori(0, CE // L, it)
        return jnp.int32(0)

    _fori(0, EQ // CE, chunk)
    pltpu.sync_copy(acc, o2p_hbm.at[q, g])


def _edges2(src, dst, h2t, as_h, ad_h):
    return pl.kernel(
        _edges2_body,
        out_type=jax.ShapeDtypeStruct((4, C, NPAD), jnp.float32),
        mesh=_mesh(),
        scratch_types=[
            pltpu.VMEM((NPAD,), jnp.float32),
            pltpu.VMEM((NPAD,), jnp.float32),
            pltpu.VMEM((NPAD,), jnp.float32),
            pltpu.VMEM((NPAD,), jnp.float32),
            pltpu.VMEM((NPAD,), jnp.float32),
            pltpu.VMEM((CE,), jnp.int32),
            pltpu.VMEM((CE,), jnp.int32),
            pltpu.VMEM((NPAD // 16,), jnp.float32),
            pltpu.VMEM((NPAD // 16,), jnp.float32),
            pltpu.VMEM_SHARED((8, NPAD), jnp.float32),
        ],
        compiler_params=_SC_PARAMS,
    )(src, dst, h2t, as_h, ad_h)


def _final_body(o2p_ref, b2_ref, out_ref):
    s = jnp.sum(o2p_ref[...], axis=0) + b2_ref[...]
    out_ref[...] = s.T


def _final(o2p, b2):
    BN = 512
    return pl.pallas_call(
        _final_body,
        grid=(NPAD // BN,),
        in_specs=[
            pl.BlockSpec((4, C, BN), lambda i: (_Z, _Z, i)),
            pl.BlockSpec((C, 1), lambda i: (_Z, _Z)),
        ],
        out_specs=pl.BlockSpec((BN, C), lambda i: (i, _Z)),
        out_shape=jax.ShapeDtypeStruct((NPAD, C), jnp.float32),
    )(o2p, b2.reshape(C, 1))


@jax.jit
def kernel(x, edge_index, W1, a1_src, a1_dst, b1, W2, a2_src, a2_dst, b2):
    src = edge_index[0].astype(jnp.int32)
    dst = edge_index[1].astype(jnp.int32)
    x_pad = jnp.pad(x, ((0, NPAD - N), (0, 0)))

    ht, as1, ad1 = _dense1(x_pad, W1, a1_src, a1_dst)
    o1t = _edges1(src, dst, ht, as1.reshape(NPAD), ad1.reshape(NPAD))
    h2t, as2, ad2 = _dense2(o1t, b1, W2, a2_src, a2_dst)
    o2p = _edges2(src, dst, h2t, as2.reshape(NPAD), ad2.reshape(NPAD))
    out = _final(o2p, b2)
    return out[:N]


# speedup vs baseline: 17.2248x; 1.1384x over previous
import functools

import jax
import jax.numpy as jnp
import numpy as np
from jax import lax
from jax.experimental import pallas as pl
from jax.experimental.pallas import tpu as pltpu
from jax.experimental.pallas import tpu_sc as plsc

N = 10000
NPAD = 10240
E = 320000
FIN = 128
HID = 128
C = 8
L = 16
CE = 2000
NSLICE = E // 16
NEG_SLOPE = 0.2

_SC_PARAMS = pltpu.CompilerParams(needs_layout_passes=False)


def _mesh():
    return plsc.VectorSubcoreMesh(core_axis_name="c", subcore_axis_name="s")


def _i32(v):
    return lax.convert_element_type(v, jnp.int32)


def _fori(lo, hi, body):
    lax.fori_loop(jnp.int32(lo), jnp.int32(hi), body, jnp.int32(0))


_Z = np.int32(0)


def _lrelu(z):
    return jnp.where(z > 0, z, NEG_SLOPE * z)


def _dense1_body(x_ref, w_ref, avs_ref, avd_ref, ht_ref, as_ref, ad_ref):
    ht = lax.dot_general(w_ref[...], x_ref[...], (((0,), (1,)), ((), ())),
                         preferred_element_type=jnp.float32)
    ht_ref[...] = ht
    as_ref[...] = lax.dot_general(avs_ref[...], ht, (((0,), (0,)), ((), ())),
                                  preferred_element_type=jnp.float32)
    ad_ref[...] = lax.dot_general(avd_ref[...], ht, (((0,), (0,)), ((), ())),
                                  preferred_element_type=jnp.float32)


def _dense1(x_pad, W1, a1_src, a1_dst):
    BN = 512
    return pl.pallas_call(
        _dense1_body,
        grid=(NPAD // BN,),
        in_specs=[
            pl.BlockSpec((BN, FIN), lambda i: (i, _Z)),
            pl.BlockSpec((FIN, HID), lambda i: (_Z, _Z)),
            pl.BlockSpec((HID, 1), lambda i: (_Z, _Z)),
            pl.BlockSpec((HID, 1), lambda i: (_Z, _Z)),
        ],
        out_specs=[
            pl.BlockSpec((HID, BN), lambda i: (_Z, i)),
            pl.BlockSpec((1, BN), lambda i: (_Z, i)),
            pl.BlockSpec((1, BN), lambda i: (_Z, i)),
        ],
        out_shape=[
            jax.ShapeDtypeStruct((HID, NPAD), jnp.float32),
            jax.ShapeDtypeStruct((1, NPAD), jnp.float32),
            jax.ShapeDtypeStruct((1, NPAD), jnp.float32),
        ],
    )(x_pad, W1, a1_src.reshape(HID, 1), a1_dst.reshape(HID, 1))


def _table_max(tab):
    def body(i, m):
        return jnp.maximum(m, tab[pl.ds(i * L, L)])
    mvec = lax.fori_loop(jnp.int32(0), jnp.int32(NPAD // L), body,
                         jnp.full((L,), -1e30, jnp.float32))
    return jnp.max(mvec)


def _zero_1d(ref):
    def body(i, _):
        ref[pl.ds(i * L, L)] = jnp.zeros((L,), jnp.float32)
        return jnp.int32(0)
    _fori(0, NPAD // L, body)


def _edge_stream(src_hbm, dst_hbm, bufs, sems, base, nch, inner):
    (sb0, sb1), (db0, db1) = bufs
    (ss0, ss1), (sd0, sd1) = sems
    sbufs, dbufs = [sb0, sb1], [db0, db1]
    ssems, dsems = [ss0, ss1], [sd0, sd1]

    def start(g, b):
        off = base + g * CE
        pltpu.async_copy(src_hbm.at[pl.ds(off, CE)], sbufs[b], ssems[b])
        pltpu.async_copy(dst_hbm.at[pl.ds(off, CE)], dbufs[b], dsems[b])

    def wait(b):
        pltpu.make_async_copy(src_hbm.at[pl.ds(_Z, CE)], sbufs[b],
                              ssems[b]).wait()
        pltpu.make_async_copy(dst_hbm.at[pl.ds(_Z, CE)], dbufs[b],
                              dsems[b]).wait()

    start(jnp.int32(0), 0)
    start(jnp.int32(1), 1)

    def outer(i, _):
        for b in range(2):
            g = i * 2 + b
            wait(b)
            inner(sbufs[b], dbufs[b])

            @pl.when(g + 2 < nch)
            def _():
                start(g + 2, b)

        return jnp.int32(0)

    _fori(0, nch // 2, outer)


def _denominators(sid, src_hbm, dst_hbm, as_t, ad_t, den_t, bufs, sems,
                  stage, acc640, sp_part, smax):
    _zero_1d(den_t)

    def inner(sbuf, dbuf):
        def it(i, _):
            sv = sbuf[pl.ds(i * L, L)]
            dv = dbuf[pl.ds(i * L, L)]
            a_s = plsc.load_gather(as_t, [sv])
            a_d = plsc.load_gather(ad_t, [dv])
            e = _lrelu(a_s + a_d)
            m = _lrelu(smax + a_d)
            plsc.addupdate_scatter(den_t, [dv], jnp.exp(e - m))
            return jnp.int32(0)

        _fori(0, CE // L, it)

    _edge_stream(src_hbm, dst_hbm, bufs, sems, sid * NSLICE,
                 NSLICE // CE, inner)

    CHK = NPAD // 16
    base = sid * CHK

    def addrows(lo, hi, first):
        def comb(k, _):
            pltpu.sync_copy(sp_part.at[k, pl.ds(base, CHK)], stage)

            def addit(i, _):
                acc640[pl.ds(i * L, L)] = (acc640[pl.ds(i * L, L)]
                                           + stage[pl.ds(i * L, L)])
                return jnp.int32(0)

            _fori(0, CHK // L, addit)
            return jnp.int32(0)

        if first:
            pltpu.sync_copy(sp_part.at[_i32(0), pl.ds(base, CHK)], acc640)
            _fori(1, hi, comb)
        else:
            _fori(lo, hi, comb)

    @pl.when(sid < 8)
    def _():
        pltpu.sync_copy(den_t, sp_part.at[sid])

    plsc.subcore_barrier()
    addrows(0, 8, True)
    plsc.subcore_barrier()

    @pl.when(sid >= 8)
    def _():
        pltpu.sync_copy(den_t, sp_part.at[sid - 8])

    plsc.subcore_barrier()
    addrows(0, 8, False)
    plsc.subcore_barrier()
    pltpu.sync_copy(acc640, sp_part.at[_i32(0), pl.ds(base, CHK)])
    plsc.subcore_barrier()
    pltpu.sync_copy(sp_part.at[_i32(0)], den_t)


def _edge_alpha(as_t, ad_t, den_t, sv, dv, smax):
    a_s = plsc.load_gather(as_t, [sv])
    a_d = plsc.load_gather(ad_t, [dv])
    den = plsc.load_gather(den_t, [dv])
    e = _lrelu(a_s + a_d)
    m = _lrelu(smax + a_d)
    return jnp.exp(e - m) / (den + 1e-16)


def _edges1_body(src_hbm, dst_hbm, ht_hbm, as_hbm, ad_hbm, o1t_hbm,
                 as_t, ad_t, den_t, tbl, acc, sb0, sb1, db0, db1,
                 stage, acc640, ss0, ss1, sd0, sd1, sp_part):
    bufs = ((sb0, sb1), (db0, db1))
    sems = ((ss0, ss1), (sd0, sd1))
    cid = _i32(lax.axis_index("c"))
    sid = _i32(lax.axis_index("s"))
    tid = cid * 16 + sid

    pltpu.sync_copy(as_hbm, as_t)
    pltpu.sync_copy(ad_hbm, ad_t)
    smax = _table_max(as_t)

    _denominators(sid, src_hbm, dst_hbm, as_t, ad_t, den_t, bufs, sems,
                  stage, acc640, sp_part, smax)

    pltpu.sync_copy(ht_hbm.at[pl.ds(tid * 4, 4)], tbl)

    jvs = [jnp.full((L,), j, jnp.int32) for j in range(4)]
    lanes = jnp.arange(L, dtype=jnp.int32)
    zvec = jnp.zeros((L,), jnp.float32)

    def zrow(i, _):
        cols = lanes + i * L
        for j in range(4):
            plsc.store_scatter(acc, [jvs[j], cols], zvec)
        return jnp.int32(0)

    _fori(0, NPAD // L, zrow)

    def inner(sbuf, dbuf):
        def it(i, _):
            sv = sbuf[pl.ds(i * L, L)]
            dv = dbuf[pl.ds(i * L, L)]
            alpha = _edge_alpha(as_t, ad_t, den_t, sv, dv, smax)
            for j in range(4):
                tv = plsc.load_gather(tbl, [jvs[j], sv])
                plsc.addupdate_scatter(acc, [jvs[j], dv], tv * alpha)
            return jnp.int32(0)

        _fori(0, CE // L, it)

    _edge_stream(src_hbm, dst_hbm, bufs, sems, _Z * CE, E // CE, inner)
    pltpu.sync_copy(acc, o1t_hbm.at[pl.ds(tid * 4, 4)])


def _edges1(src, dst, ht, as_h, ad_h):
    return pl.kernel(
        _edges1_body,
        out_type=jax.ShapeDtypeStruct((HID, NPAD), jnp.float32),
        mesh=_mesh(),
        scratch_types=[
            pltpu.VMEM((NPAD,), jnp.float32),
            pltpu.VMEM((NPAD,), jnp.float32),
            pltpu.VMEM((NPAD,), jnp.float32),
            pltpu.VMEM((4, NPAD), jnp.float32),
            pltpu.VMEM((4, NPAD), jnp.float32),
            pltpu.VMEM((CE,), jnp.int32),
            pltpu.VMEM((CE,), jnp.int32),
            pltpu.VMEM((CE,), jnp.int32),
            pltpu.VMEM((CE,), jnp.int32),
            pltpu.VMEM((NPAD // 16,), jnp.float32),
            pltpu.VMEM((NPAD // 16,), jnp.float32),
            pltpu.SemaphoreType.DMA,
            pltpu.SemaphoreType.DMA,
            pltpu.SemaphoreType.DMA,
            pltpu.SemaphoreType.DMA,
            pltpu.VMEM_SHARED((8, NPAD), jnp.float32),
        ],
        compiler_params=_SC_PARAMS,
    )(src, dst, ht, as_h, ad_h)


def _dense2_body(o1t_ref, b1_ref, w2_ref, avs_ref, avd_ref,
                 h2t_ref, as2_ref, ad2_ref):
    h1 = jnp.maximum(o1t_ref[...] + b1_ref[...], 0.0)
    h2t = lax.dot_general(w2_ref[...], h1, (((0,), (0,)), ((), ())),
                          preferred_element_type=jnp.float32)
    h2t_ref[...] = h2t
    as2_ref[...] = lax.dot_general(avs_ref[...], h2t, (((0,), (0,)), ((), ())),
                                   preferred_element_type=jnp.float32)
    ad2_ref[...] = lax.dot_general(avd_ref[...], h2t, (((0,), (0,)), ((), ())),
                                   preferred_element_type=jnp.float32)


def _dense2(o1t, b1, W2, a2_src, a2_dst):
    BN = 512
    return pl.pallas_call(
        _dense2_body,
        grid=(NPAD // BN,),
        in_specs=[
            pl.BlockSpec((HID, BN), lambda i: (_Z, i)),
            pl.BlockSpec((HID, 1), lambda i: (_Z, _Z)),
            pl.BlockSpec((HID, C), lambda i: (_Z, _Z)),
            pl.BlockSpec((C, 1), lambda i: (_Z, _Z)),
            pl.BlockSpec((C, 1), lambda i: (_Z, _Z)),
        ],
        out_specs=[
            pl.BlockSpec((C, BN), lambda i: (_Z, i)),
            pl.BlockSpec((1, BN), lambda i: (_Z, i)),
            pl.BlockSpec((1, BN), lambda i: (_Z, i)),
        ],
        out_shape=[
            jax.ShapeDtypeStruct((C, NPAD), jnp.float32),
            jax.ShapeDtypeStruct((1, NPAD), jnp.float32),
            jax.ShapeDtypeStruct((1, NPAD), jnp.float32),
        ],
    )(o1t, b1.reshape(HID, 1), W2, a2_src.reshape(C, 1), a2_dst.reshape(C, 1))


EQ = E // 4


def _edges2_body(src_hbm, dst_hbm, h2t_hbm, as_hbm, ad_hbm, o2p_hbm,
                 as_t, ad_t, den_t, tbl, acc, sb0, sb1, db0, db1,
                 stage, acc640, ss0, ss1, sd0, sd1, sp_part):
    bufs = ((sb0, sb1), (db0, db1))
    sems = ((ss0, ss1), (sd0, sd1))
    cid = _i32(lax.axis_index("c"))
    sid = _i32(lax.axis_index("s"))
    tid = cid * 16 + sid
    q = tid // C
    g = tid % C

    pltpu.sync_copy(as_hbm, as_t)
    pltpu.sync_copy(ad_hbm, ad_t)
    smax = _table_max(as_t)

    _denominators(sid, src_hbm, dst_hbm, as_t, ad_t, den_t, bufs, sems,
                  stage, acc640, sp_part, smax)

    pltpu.sync_copy(h2t_hbm.at[g], tbl)
    _zero_1d(acc)

    def inner(sbuf, dbuf):
        def it(i, _):
            sv = sbuf[pl.ds(i * L, L)]
            dv = dbuf[pl.ds(i * L, L)]
            alpha = _edge_alpha(as_t, ad_t, den_t, sv, dv, smax)
            tv = plsc.load_gather(tbl, [sv])
            plsc.addupdate_scatter(acc, [dv], tv * alpha)
            return jnp.int32(0)

        _fori(0, CE // L, it)

    _edge_stream(src_hbm, dst_hbm, bufs, sems, q * EQ, EQ // CE, inner)
    pltpu.sync_copy(acc, o2p_hbm.at[q, g])


def _edges2(src, dst, h2t, as_h, ad_h):
    return pl.kernel(
        _edges2_body,
        out_type=jax.ShapeDtypeStruct((4, C, NPAD), jnp.float32),
        mesh=_mesh(),
        scratch_types=[
            pltpu.VMEM((NPAD,), jnp.float32),
            pltpu.VMEM((NPAD,), jnp.float32),
            pltpu.VMEM((NPAD,), jnp.float32),
            pltpu.VMEM((NPAD,), jnp.float32),
            pltpu.VMEM((NPAD,), jnp.float32),
            pltpu.VMEM((CE,), jnp.int32),
            pltpu.VMEM((CE,), jnp.int32),
            pltpu.VMEM((CE,), jnp.int32),
            pltpu.VMEM((CE,), jnp.int32),
            pltpu.VMEM((NPAD // 16,), jnp.float32),
            pltpu.VMEM((NPAD // 16,), jnp.float32),
            pltpu.SemaphoreType.DMA,
            pltpu.SemaphoreType.DMA,
            pltpu.SemaphoreType.DMA,
            pltpu.SemaphoreType.DMA,
            pltpu.VMEM_SHARED((8, NPAD), jnp.float32),
        ],
        compiler_params=_SC_PARAMS,
    )(src, dst, h2t, as_h, ad_h)


def _final_body(o2p_ref, b2_ref, out_ref):
    s = jnp.sum(o2p_ref[...], axis=0) + b2_ref[...]
    out_ref[...] = s.T


def _final(o2p, b2):
    BN = 512
    return pl.pallas_call(
        _final_body,
        grid=(NPAD // BN,),
        in_specs=[
            pl.BlockSpec((4, C, BN), lambda i: (_Z, _Z, i)),
            pl.BlockSpec((C, 1), lambda i: (_Z, _Z)),
        ],
        out_specs=pl.BlockSpec((BN, C), lambda i: (i, _Z)),
        out_shape=jax.ShapeDtypeStruct((NPAD, C), jnp.float32),
    )(o2p, b2.reshape(C, 1))


@jax.jit
def kernel(x, edge_index, W1, a1_src, a1_dst, b1, W2, a2_src, a2_dst, b2):
    src = edge_index[0].astype(jnp.int32)
    dst = edge_index[1].astype(jnp.int32)
    x_pad = jnp.pad(x, ((0, NPAD - N), (0, 0)))

    ht, as1, ad1 = _dense1(x_pad, W1, a1_src, a1_dst)
    o1t = _edges1(src, dst, ht, as1.reshape(NPAD), ad1.reshape(NPAD))
    h2t, as2, ad2 = _dense2(o1t, b1, W2, a2_src, a2_dst)
    o2p = _edges2(src, dst, h2t, as2.reshape(NPAD), ad2.reshape(NPAD))
    out = _final(o2p, b2)
    return out[:N]


# speedup vs baseline: 47.0811x; 2.7333x over previous
import functools

import jax
import jax.numpy as jnp
import numpy as np
from jax import lax
from jax.experimental import pallas as pl
from jax.experimental.pallas import tpu as pltpu
from jax.experimental.pallas import tpu_sc as plsc

N = 10000
NPAD = 10240
E = 320000
FIN = 128
HID = 128
C = 8
L = 16
CE = 2000
NSLICE = E // 16
NEG_SLOPE = 0.2

_SC_PARAMS = pltpu.CompilerParams(needs_layout_passes=False)


def _mesh():
    return plsc.VectorSubcoreMesh(core_axis_name="c", subcore_axis_name="s")


def _i32(v):
    return lax.convert_element_type(v, jnp.int32)


def _fori(lo, hi, body):
    lax.fori_loop(jnp.int32(lo), jnp.int32(hi), body, jnp.int32(0))


_Z = np.int32(0)


def _lrelu(z):
    return jnp.where(z > 0, z, NEG_SLOPE * z)


def _dense1_body(x_ref, w_ref, avs_ref, avd_ref, ht_ref, as_ref, ad_ref):
    ht = lax.dot_general(w_ref[...], x_ref[...], (((0,), (1,)), ((), ())),
                         preferred_element_type=jnp.float32)
    ht_ref[...] = ht
    as_ref[...] = lax.dot_general(avs_ref[...], ht, (((0,), (0,)), ((), ())),
                                  preferred_element_type=jnp.float32)
    ad_ref[...] = lax.dot_general(avd_ref[...], ht, (((0,), (0,)), ((), ())),
                                  preferred_element_type=jnp.float32)


def _dense1(x_pad, W1, a1_src, a1_dst):
    BN = 512
    return pl.pallas_call(
        _dense1_body,
        grid=(NPAD // BN,),
        in_specs=[
            pl.BlockSpec((BN, FIN), lambda i: (i, _Z)),
            pl.BlockSpec((FIN, HID), lambda i: (_Z, _Z)),
            pl.BlockSpec((HID, 1), lambda i: (_Z, _Z)),
            pl.BlockSpec((HID, 1), lambda i: (_Z, _Z)),
        ],
        out_specs=[
            pl.BlockSpec((HID, BN), lambda i: (_Z, i)),
            pl.BlockSpec((1, BN), lambda i: (_Z, i)),
            pl.BlockSpec((1, BN), lambda i: (_Z, i)),
        ],
        out_shape=[
            jax.ShapeDtypeStruct((HID, NPAD), jnp.float32),
            jax.ShapeDtypeStruct((1, NPAD), jnp.float32),
            jax.ShapeDtypeStruct((1, NPAD), jnp.float32),
        ],
    )(x_pad, W1, a1_src.reshape(HID, 1), a1_dst.reshape(HID, 1))


def _table_max(tab):
    def body(i, m):
        return jnp.maximum(m, tab[pl.ds(i * L, L)])
    mvec = lax.fori_loop(jnp.int32(0), jnp.int32(NPAD // L), body,
                         jnp.full((L,), -1e30, jnp.float32))
    return jnp.max(mvec)


def _zero_1d(ref):
    @plsc.parallel_loop(jnp.int32(0), jnp.int32(NPAD // L), step=jnp.int32(1), unroll=8)
    def body(i):
        ref[pl.ds(i * L, L)] = jnp.zeros((L,), jnp.float32)


def _edge_stream(src_hbm, dst_hbm, bufs, sems, base, nch, inner):
    (sb0, sb1), (db0, db1) = bufs
    (ss0, ss1), (sd0, sd1) = sems
    sbufs, dbufs = [sb0, sb1], [db0, db1]
    ssems, dsems = [ss0, ss1], [sd0, sd1]

    def start(g, b):
        off = base + g * CE
        pltpu.async_copy(src_hbm.at[pl.ds(off, CE)], sbufs[b], ssems[b])
        pltpu.async_copy(dst_hbm.at[pl.ds(off, CE)], dbufs[b], dsems[b])

    def wait(b):
        pltpu.make_async_copy(src_hbm.at[pl.ds(_Z, CE)], sbufs[b],
                              ssems[b]).wait()
        pltpu.make_async_copy(dst_hbm.at[pl.ds(_Z, CE)], dbufs[b],
                              dsems[b]).wait()

    start(jnp.int32(0), 0)
    start(jnp.int32(1), 1)

    def outer(i, _):
        for b in range(2):
            g = i * 2 + b
            wait(b)
            inner(sbufs[b], dbufs[b])

            @pl.when(g + 2 < nch)
            def _():
                start(g + 2, b)

        return jnp.int32(0)

    _fori(0, nch // 2, outer)


def _denominators(sid, src_hbm, dst_hbm, as_t, ad_t, den_t, bufs, sems,
                  stage, acc640, sp_part, smax):
    _zero_1d(den_t)

    def inner(sbuf, dbuf):
        @plsc.parallel_loop(jnp.int32(0), jnp.int32(CE // L), step=jnp.int32(1), unroll=8)
        def it(i):
            sv = sbuf[pl.ds(i * L, L)]
            dv = dbuf[pl.ds(i * L, L)]
            a_s = plsc.load_gather(as_t, [sv])
            a_d = plsc.load_gather(ad_t, [dv])
            e = _lrelu(a_s + a_d)
            m = _lrelu(smax + a_d)
            plsc.addupdate_scatter(den_t, [dv], jnp.exp(e - m))

    _edge_stream(src_hbm, dst_hbm, bufs, sems, sid * NSLICE,
                 NSLICE // CE, inner)

    CHK = NPAD // 16
    base = sid * CHK

    def addrows(lo, hi, first):
        def comb(k, _):
            pltpu.sync_copy(sp_part.at[k, pl.ds(base, CHK)], stage)

            @plsc.parallel_loop(jnp.int32(0), jnp.int32(CHK // L), step=jnp.int32(1), unroll=8)
            def addit(i):
                acc640[pl.ds(i * L, L)] = (acc640[pl.ds(i * L, L)]
                                           + stage[pl.ds(i * L, L)])
            return jnp.int32(0)

        if first:
            pltpu.sync_copy(sp_part.at[_i32(0), pl.ds(base, CHK)], acc640)
            _fori(1, hi, comb)
        else:
            _fori(lo, hi, comb)

    @pl.when(sid < 8)
    def _():
        pltpu.sync_copy(den_t, sp_part.at[sid])

    plsc.subcore_barrier()
    addrows(0, 8, True)
    plsc.subcore_barrier()

    @pl.when(sid >= 8)
    def _():
        pltpu.sync_copy(den_t, sp_part.at[sid - 8])

    plsc.subcore_barrier()
    addrows(0, 8, False)
    plsc.subcore_barrier()
    pltpu.sync_copy(acc640, sp_part.at[_i32(0), pl.ds(base, CHK)])
    plsc.subcore_barrier()
    pltpu.sync_copy(sp_part.at[_i32(0)], den_t)


def _edge_alpha(as_t, ad_t, den_t, sv, dv, smax):
    a_s = plsc.load_gather(as_t, [sv])
    a_d = plsc.load_gather(ad_t, [dv])
    den = plsc.load_gather(den_t, [dv])
    e = _lrelu(a_s + a_d)
    m = _lrelu(smax + a_d)
    return jnp.exp(e - m) / (den + 1e-16)


def _edges1_body(src_hbm, dst_hbm, ht_hbm, as_hbm, ad_hbm, o1t_hbm,
                 as_t, ad_t, den_t, tbl, acc, sb0, sb1, db0, db1,
                 stage, acc640, ss0, ss1, sd0, sd1, sp_part):
    bufs = ((sb0, sb1), (db0, db1))
    sems = ((ss0, ss1), (sd0, sd1))
    cid = _i32(lax.axis_index("c"))
    sid = _i32(lax.axis_index("s"))
    tid = cid * 16 + sid

    pltpu.sync_copy(as_hbm, as_t)
    pltpu.sync_copy(ad_hbm, ad_t)
    smax = _table_max(as_t)

    _denominators(sid, src_hbm, dst_hbm, as_t, ad_t, den_t, bufs, sems,
                  stage, acc640, sp_part, smax)

    pltpu.sync_copy(ht_hbm.at[pl.ds(tid * 4, 4)], tbl)

    jvs = [jnp.full((L,), j, jnp.int32) for j in range(4)]
    lanes = jnp.arange(L, dtype=jnp.int32)
    zvec = jnp.zeros((L,), jnp.float32)

    @plsc.parallel_loop(jnp.int32(0), jnp.int32(NPAD // L), step=jnp.int32(1), unroll=8)
    def zrow(i):
        cols = lanes + i * L
        for j in range(4):
            plsc.store_scatter(acc, [jvs[j], cols], zvec)

    def inner(sbuf, dbuf):
        @plsc.parallel_loop(jnp.int32(0), jnp.int32(CE // L), step=jnp.int32(1), unroll=4)
        def it(i):
            sv = sbuf[pl.ds(i * L, L)]
            dv = dbuf[pl.ds(i * L, L)]
            alpha = _edge_alpha(as_t, ad_t, den_t, sv, dv, smax)
            for j in range(4):
                tv = plsc.load_gather(tbl, [jvs[j], sv])
                plsc.addupdate_scatter(acc, [jvs[j], dv], tv * alpha)

    _edge_stream(src_hbm, dst_hbm, bufs, sems, _Z * CE, E // CE, inner)
    pltpu.sync_copy(acc, o1t_hbm.at[pl.ds(tid * 4, 4)])


def _edges1(src, dst, ht, as_h, ad_h):
    return pl.kernel(
        _edges1_body,
        out_type=jax.ShapeDtypeStruct((HID, NPAD), jnp.float32),
        mesh=_mesh(),
        scratch_types=[
            pltpu.VMEM((NPAD,), jnp.float32),
            pltpu.VMEM((NPAD,), jnp.float32),
            pltpu.VMEM((NPAD,), jnp.float32),
            pltpu.VMEM((4, NPAD), jnp.float32),
            pltpu.VMEM((4, NPAD), jnp.float32),
            pltpu.VMEM((CE,), jnp.int32),
            pltpu.VMEM((CE,), jnp.int32),
            pltpu.VMEM((CE,), jnp.int32),
            pltpu.VMEM((CE,), jnp.int32),
            pltpu.VMEM((NPAD // 16,), jnp.float32),
            pltpu.VMEM((NPAD // 16,), jnp.float32),
            pltpu.SemaphoreType.DMA,
            pltpu.SemaphoreType.DMA,
            pltpu.SemaphoreType.DMA,
            pltpu.SemaphoreType.DMA,
            pltpu.VMEM_SHARED((8, NPAD), jnp.float32),
        ],
        compiler_params=_SC_PARAMS,
    )(src, dst, ht, as_h, ad_h)


def _dense2_body(o1t_ref, b1_ref, w2_ref, avs_ref, avd_ref,
                 h2t_ref, as2_ref, ad2_ref):
    h1 = jnp.maximum(o1t_ref[...] + b1_ref[...], 0.0)
    h2t = lax.dot_general(w2_ref[...], h1, (((0,), (0,)), ((), ())),
                          preferred_element_type=jnp.float32)
    h2t_ref[...] = h2t
    as2_ref[...] = lax.dot_general(avs_ref[...], h2t, (((0,), (0,)), ((), ())),
                                   preferred_element_type=jnp.float32)
    ad2_ref[...] = lax.dot_general(avd_ref[...], h2t, (((0,), (0,)), ((), ())),
                                   preferred_element_type=jnp.float32)


def _dense2(o1t, b1, W2, a2_src, a2_dst):
    BN = 512
    return pl.pallas_call(
        _dense2_body,
        grid=(NPAD // BN,),
        in_specs=[
            pl.BlockSpec((HID, BN), lambda i: (_Z, i)),
            pl.BlockSpec((HID, 1), lambda i: (_Z, _Z)),
            pl.BlockSpec((HID, C), lambda i: (_Z, _Z)),
            pl.BlockSpec((C, 1), lambda i: (_Z, _Z)),
            pl.BlockSpec((C, 1), lambda i: (_Z, _Z)),
        ],
        out_specs=[
            pl.BlockSpec((C, BN), lambda i: (_Z, i)),
            pl.BlockSpec((1, BN), lambda i: (_Z, i)),
            pl.BlockSpec((1, BN), lambda i: (_Z, i)),
        ],
        out_shape=[
            jax.ShapeDtypeStruct((C, NPAD), jnp.float32),
            jax.ShapeDtypeStruct((1, NPAD), jnp.float32),
            jax.ShapeDtypeStruct((1, NPAD), jnp.float32),
        ],
    )(o1t, b1.reshape(HID, 1), W2, a2_src.reshape(C, 1), a2_dst.reshape(C, 1))


EQ = E // 4


def _edges2_body(src_hbm, dst_hbm, h2t_hbm, as_hbm, ad_hbm, o2p_hbm,
                 as_t, ad_t, den_t, tbl, acc, sb0, sb1, db0, db1,
                 stage, acc640, ss0, ss1, sd0, sd1, sp_part):
    bufs = ((sb0, sb1), (db0, db1))
    sems = ((ss0, ss1), (sd0, sd1))
    cid = _i32(lax.axis_index("c"))
    sid = _i32(lax.axis_index("s"))
    tid = cid * 16 + sid
    q = tid // C
    g = tid % C

    pltpu.sync_copy(as_hbm, as_t)
    pltpu.sync_copy(ad_hbm, ad_t)
    smax = _table_max(as_t)

    _denominators(sid, src_hbm, dst_hbm, as_t, ad_t, den_t, bufs, sems,
                  stage, acc640, sp_part, smax)

    pltpu.sync_copy(h2t_hbm.at[g], tbl)
    _zero_1d(acc)

    def inner(sbuf, dbuf):
        @plsc.parallel_loop(jnp.int32(0), jnp.int32(CE // L), step=jnp.int32(1), unroll=8)
        def it(i):
            sv = sbuf[pl.ds(i * L, L)]
            dv = dbuf[pl.ds(i * L, L)]
            alpha = _edge_alpha(as_t, ad_t, den_t, sv, dv, smax)
            tv = plsc.load_gather(tbl, [sv])
            plsc.addupdate_scatter(acc, [dv], tv * alpha)

    _edge_stream(src_hbm, dst_hbm, bufs, sems, q * EQ, EQ // CE, inner)
    pltpu.sync_copy(acc, o2p_hbm.at[q, g])


def _edges2(src, dst, h2t, as_h, ad_h):
    return pl.kernel(
        _edges2_body,
        out_type=jax.ShapeDtypeStruct((4, C, NPAD), jnp.float32),
        mesh=_mesh(),
        scratch_types=[
            pltpu.VMEM((NPAD,), jnp.float32),
            pltpu.VMEM((NPAD,), jnp.float32),
            pltpu.VMEM((NPAD,), jnp.float32),
            pltpu.VMEM((NPAD,), jnp.float32),
            pltpu.VMEM((NPAD,), jnp.float32),
            pltpu.VMEM((CE,), jnp.int32),
            pltpu.VMEM((CE,), jnp.int32),
            pltpu.VMEM((CE,), jnp.int32),
            pltpu.VMEM((CE,), jnp.int32),
            pltpu.VMEM((NPAD // 16,), jnp.float32),
            pltpu.VMEM((NPAD // 16,), jnp.float32),
            pltpu.SemaphoreType.DMA,
            pltpu.SemaphoreType.DMA,
            pltpu.SemaphoreType.DMA,
            pltpu.SemaphoreType.DMA,
            pltpu.VMEM_SHARED((8, NPAD), jnp.float32),
        ],
        compiler_params=_SC_PARAMS,
    )(src, dst, h2t, as_h, ad_h)


def _final_body(o2p_ref, b2_ref, out_ref):
    s = jnp.sum(o2p_ref[...], axis=0) + b2_ref[...]
    out_ref[...] = s.T


def _final(o2p, b2):
    BN = 512
    return pl.pallas_call(
        _final_body,
        grid=(NPAD // BN,),
        in_specs=[
            pl.BlockSpec((4, C, BN), lambda i: (_Z, _Z, i)),
            pl.BlockSpec((C, 1), lambda i: (_Z, _Z)),
        ],
        out_specs=pl.BlockSpec((BN, C), lambda i: (i, _Z)),
        out_shape=jax.ShapeDtypeStruct((NPAD, C), jnp.float32),
    )(o2p, b2.reshape(C, 1))


@jax.jit
def kernel(x, edge_index, W1, a1_src, a1_dst, b1, W2, a2_src, a2_dst, b2):
    src = edge_index[0].astype(jnp.int32)
    dst = edge_index[1].astype(jnp.int32)
    x_pad = jnp.pad(x, ((0, NPAD - N), (0, 0)))

    ht, as1, ad1 = _dense1(x_pad, W1, a1_src, a1_dst)
    o1t = _edges1(src, dst, ht, as1.reshape(NPAD), ad1.reshape(NPAD))
    h2t, as2, ad2 = _dense2(o1t, b1, W2, a2_src, a2_dst)
    o2p = _edges2(src, dst, h2t, as2.reshape(NPAD), ad2.reshape(NPAD))
    out = _final(o2p, b2)
    return out[:N]
